# Initial kernel scaffold; baseline (speedup 1.0000x reference)
import jax
import jax.numpy as jnp
from jax.experimental import pallas as pl


def kernel(x, scale_idx, scale_embeddings, W):
    raise NotImplementedError("write your pallas kernel here")


# speedup vs baseline: 4.8197x; 4.8197x over previous
import functools

import jax
import jax.numpy as jnp
from jax import lax
from jax.experimental import pallas as pl
from jax.experimental.pallas import tpu as pltpu

TOP_K = 8
_BT = 512


def _router_block(si_ref, emb_ref, wst_ref, x_ref, wxt_ref,
                  disp_ref, probs_ref, sel_ref, wts_ref):
    e = probs_ref.shape[-1]
    si = si_ref[0]
    emb = emb_ref[pl.ds(si, 1), :]
    bias = jnp.dot(emb, wst_ref[:, :], preferred_element_type=jnp.float32)
    logits = jnp.dot(x_ref[:, :], wxt_ref[:, :],
                     preferred_element_type=jnp.float32) + bias

    m = jnp.max(logits, axis=1, keepdims=True)
    ex = jnp.exp(logits - m)
    probs = ex / jnp.sum(ex, axis=1, keepdims=True)
    probs_ref[:, :] = probs

    col = lax.broadcasted_iota(jnp.int32, probs.shape, 1)
    work = probs
    topmask = jnp.zeros(probs.shape, dtype=jnp.bool_)
    vals, idxs = [], []
    for _ in range(TOP_K):
        mx = jnp.max(work, axis=1, keepdims=True)
        amax = jnp.min(jnp.where(work == mx, col, e), axis=1, keepdims=True)
        onehot = col == amax
        vals.append(mx)
        idxs.append(amax)
        topmask = jnp.logical_or(topmask, onehot)
        work = jnp.where(onehot, -1.0, work)

    inv = 1.0 / functools.reduce(jnp.add, vals)
    wts_ref[:, :] = jnp.concatenate(vals, axis=1) * inv
    sel_ref[:, :] = jnp.concatenate(idxs, axis=1)
    disp_ref[:, :] = jnp.where(topmask, probs * inv, 0.0)


def kernel(x, scale_idx, scale_embeddings, W):
    B, S, D = x.shape
    T = B * S
    E, DW = W.shape
    Ds = DW - D
    xf = x.reshape(T, D)
    wxt = W[:, :D].T
    wst = W[:, D:].T
    pad = (-scale_embeddings.shape[0]) % 8
    emb = jnp.pad(scale_embeddings, ((0, pad), (0, 0)))
    si = jnp.asarray(scale_idx, jnp.int32).reshape((1,))

    grid = (T // _BT,)
    disp, probs, sel, wts = pl.pallas_call(
        _router_block,
        grid=grid,
        in_specs=[
            pl.BlockSpec(memory_space=pltpu.SMEM),
            pl.BlockSpec(emb.shape, lambda i: (0, 0)),
            pl.BlockSpec((Ds, E), lambda i: (0, 0)),
            pl.BlockSpec((_BT, D), lambda i: (i, 0)),
            pl.BlockSpec((D, E), lambda i: (0, 0)),
        ],
        out_specs=[
            pl.BlockSpec((_BT, E), lambda i: (i, 0)),
            pl.BlockSpec((_BT, E), lambda i: (i, 0)),
            pl.BlockSpec((_BT, TOP_K), lambda i: (i, 0)),
            pl.BlockSpec((_BT, TOP_K), lambda i: (i, 0)),
        ],
        out_shape=[
            jax.ShapeDtypeStruct((T, E), jnp.float32),
            jax.ShapeDtypeStruct((T, E), jnp.float32),
            jax.ShapeDtypeStruct((T, TOP_K), jnp.int32),
            jax.ShapeDtypeStruct((T, TOP_K), jnp.float32),
        ],
    )(si, emb, wst, xf, wxt)

    return (disp.reshape(B, S, E), probs.reshape(B, S, E),
            sel.reshape(B, S, TOP_K), wts.reshape(B, S, TOP_K))


# speedup vs baseline: 5.6444x; 1.1711x over previous
import functools

import jax
import jax.numpy as jnp
from jax import lax
from jax.experimental import pallas as pl
from jax.experimental.pallas import tpu as pltpu

TOP_K = 8
_BT = 512


def _router_block(si_ref, emb_ref, wst_ref, x_ref, wxt_ref,
                  disp_ref, probs_ref, sel_ref, wts_ref):
    e = probs_ref.shape[-1]
    si = si_ref[0]
    emb = emb_ref[pl.ds(si, 1), :]
    bias = jnp.dot(emb, wst_ref[:, :], preferred_element_type=jnp.float32)
    logits = jnp.dot(x_ref[:, :], wxt_ref[:, :],
                     preferred_element_type=jnp.float32) + bias

    m = jnp.max(logits, axis=1, keepdims=True)
    ex = jnp.exp(logits - m)
    probs = ex / jnp.sum(ex, axis=1, keepdims=True)
    probs_ref[:, :] = probs

    col = lax.broadcasted_iota(jnp.int32, probs.shape, 1)
    kbits = (lax.bitcast_convert_type(probs, jnp.int32) & ~63) | (e - 1 - col)
    keys = lax.bitcast_convert_type(kbits, jnp.float32)
    work = keys
    vals, idxs = [], []
    mxk = None
    for _ in range(TOP_K):
        mxk = jnp.max(work, axis=1, keepdims=True)
        kb = lax.bitcast_convert_type(mxk, jnp.int32)
        idxs.append((e - 1) - (kb & 63))
        vals.append(lax.bitcast_convert_type(kb & ~63, jnp.float32))
        work = jnp.where(work == mxk, -1.0, work)

    inv = 1.0 / functools.reduce(jnp.add, vals)
    wts_ref[:, :] = jnp.concatenate(vals, axis=1) * inv
    sel_ref[:, :] = jnp.concatenate(idxs, axis=1)
    disp_ref[:, :] = jnp.where(keys >= mxk, probs * inv, 0.0)


def kernel(x, scale_idx, scale_embeddings, W):
    B, S, D = x.shape
    T = B * S
    E, DW = W.shape
    Ds = DW - D
    xf = x.reshape(T, D)
    wxt = W[:, :D].T
    wst = W[:, D:].T
    pad = (-scale_embeddings.shape[0]) % 8
    emb = jnp.pad(scale_embeddings, ((0, pad), (0, 0)))
    si = jnp.asarray(scale_idx, jnp.int32).reshape((1,))

    grid = (T // _BT,)
    disp, probs, sel, wts = pl.pallas_call(
        _router_block,
        grid=grid,
        in_specs=[
            pl.BlockSpec(memory_space=pltpu.SMEM),
            pl.BlockSpec(emb.shape, lambda i: (0, 0)),
            pl.BlockSpec((Ds, E), lambda i: (0, 0)),
            pl.BlockSpec((_BT, D), lambda i: (i, 0)),
            pl.BlockSpec((D, E), lambda i: (0, 0)),
        ],
        out_specs=[
            pl.BlockSpec((_BT, E), lambda i: (i, 0)),
            pl.BlockSpec((_BT, E), lambda i: (i, 0)),
            pl.BlockSpec((_BT, TOP_K), lambda i: (i, 0)),
            pl.BlockSpec((_BT, TOP_K), lambda i: (i, 0)),
        ],
        out_shape=[
            jax.ShapeDtypeStruct((T, E), jnp.float32),
            jax.ShapeDtypeStruct((T, E), jnp.float32),
            jax.ShapeDtypeStruct((T, TOP_K), jnp.int32),
            jax.ShapeDtypeStruct((T, TOP_K), jnp.float32),
        ],
    )(si, emb, wst, xf, wxt)

    return (disp.reshape(B, S, E), probs.reshape(B, S, E),
            sel.reshape(B, S, TOP_K), wts.reshape(B, S, TOP_K))


# speedup vs baseline: 5.8803x; 1.0418x over previous
import functools

import jax
import jax.numpy as jnp
from jax import lax
from jax.experimental import pallas as pl
from jax.experimental.pallas import tpu as pltpu

TOP_K = 8
_BT = 1024


def _router_block(si_ref, emb_ref, wst_ref, x_ref, wxt_ref,
                  disp_ref, probs_ref, sel_ref, wts_ref):
    e = probs_ref.shape[-1]
    si = si_ref[0]
    emb = emb_ref[pl.ds(si, 1), :]
    bias = jnp.dot(emb, wst_ref[:, :], preferred_element_type=jnp.float32)
    logits = jnp.dot(x_ref[:, :], wxt_ref[:, :],
                     preferred_element_type=jnp.float32) + bias

    m = jnp.max(logits, axis=1, keepdims=True)
    ex = jnp.exp(logits - m)
    probs = ex / jnp.sum(ex, axis=1, keepdims=True)
    probs_ref[:, :] = probs

    col = lax.broadcasted_iota(jnp.int32, probs.shape, 1)
    kbits = (lax.bitcast_convert_type(probs, jnp.int32) & ~63) | (e - 1 - col)
    keys = lax.bitcast_convert_type(kbits, jnp.float32)
    work = keys
    vals, idxs = [], []
    mxk = None
    for _ in range(TOP_K):
        mxk = jnp.max(work, axis=1, keepdims=True)
        kb = lax.bitcast_convert_type(mxk, jnp.int32)
        idxs.append((e - 1) - (kb & 63))
        vals.append(lax.bitcast_convert_type(kb & ~63, jnp.float32))
        work = jnp.where(work == mxk, -1.0, work)

    inv = 1.0 / functools.reduce(jnp.add, vals)
    wts_ref[:, :] = jnp.concatenate(vals, axis=1) * inv
    sel_ref[:, :] = jnp.concatenate(idxs, axis=1)
    disp_ref[:, :] = jnp.where(keys >= mxk, probs * inv, 0.0)


def kernel(x, scale_idx, scale_embeddings, W):
    B, S, D = x.shape
    T = B * S
    E, DW = W.shape
    Ds = DW - D
    xf = x.reshape(T, D)
    wxt = W[:, :D].T
    wst = W[:, D:].T
    pad = (-scale_embeddings.shape[0]) % 8
    emb = jnp.pad(scale_embeddings, ((0, pad), (0, 0)))
    si = jnp.asarray(scale_idx, jnp.int32).reshape((1,))

    grid = (T // _BT,)
    disp, probs, sel, wts = pl.pallas_call(
        _router_block,
        grid=grid,
        in_specs=[
            pl.BlockSpec(memory_space=pltpu.SMEM),
            pl.BlockSpec(emb.shape, lambda i: (0, 0)),
            pl.BlockSpec((Ds, E), lambda i: (0, 0)),
            pl.BlockSpec((_BT, D), lambda i: (i, 0)),
            pl.BlockSpec((D, E), lambda i: (0, 0)),
        ],
        out_specs=[
            pl.BlockSpec((_BT, E), lambda i: (i, 0)),
            pl.BlockSpec((_BT, E), lambda i: (i, 0)),
            pl.BlockSpec((_BT, TOP_K), lambda i: (i, 0)),
            pl.BlockSpec((_BT, TOP_K), lambda i: (i, 0)),
        ],
        out_shape=[
            jax.ShapeDtypeStruct((T, E), jnp.float32),
            jax.ShapeDtypeStruct((T, E), jnp.float32),
            jax.ShapeDtypeStruct((T, TOP_K), jnp.int32),
            jax.ShapeDtypeStruct((T, TOP_K), jnp.float32),
        ],
    )(si, emb, wst, xf, wxt)

    return (disp.reshape(B, S, E), probs.reshape(B, S, E),
            sel.reshape(B, S, TOP_K), wts.reshape(B, S, TOP_K))
